# baseline (speedup 1.0000x reference)
import functools

import jax
import jax.numpy as jnp
from jax import lax
from jax.experimental import pallas as pl
from jax.experimental.pallas import tpu as pltpu
from jax.experimental.pallas import tpu_sc as plsc

_RATIO = 0.5
_NC = 2
_NS = 16
_L = 16
_NW = _NC * _NS
_BLK = 8192


def _hist_body(x_hbm, pref_hbm, out_hbm, buf0, buf1, hist, pref_v,
               sem0, sem1, *, n_total, b_bins, sh_pref, sh_idx, dual):
    per_tile = n_total // _NW
    n_blk = per_tile // _BLK
    wid = lax.axis_index("s") * _NC + lax.axis_index("c")
    base = wid * per_tile
    nbt = (2 if dual else 1) * b_bins

    def issue(i, buf, sem):
        pltpu.async_copy(x_hbm.at[pl.ds(base + i * _BLK, _BLK)], buf, sem)

    def drain(buf, sem):
        pltpu.make_async_copy(x_hbm.at[pl.ds(base, _BLK)], buf, sem).wait()

    issue(0, buf0, sem0)

    zeros16 = jnp.zeros((_L,), jnp.int32)
    col16 = lax.iota(jnp.int32, _L)

    def zero_body(r, c):
        rv = jnp.full((_L,), r, jnp.int32)
        for t in range(8):
            plsc.store_scatter(hist, [rv, col16 + t * _L], zeros16)
        return c

    lax.fori_loop(0, nbt // 128, zero_body, 0)

    pltpu.sync_copy(pref_hbm, pref_v)
    pa = pref_v[pl.ds(0, _L)]
    pb = pref_v[pl.ds(_L, _L)]

    ones = jnp.ones((_L,), jnp.int32)
    m127 = jnp.int32(127)
    rmask = jnp.int32(b_bins // 128 - 1)
    pmask = jnp.int32(0xFFFF)
    off_b = jnp.int32(b_bins // 128)

    def consume(buf):
        def g_body(g, cc):
            for t in range(8):
                v = buf[pl.ds((g * 8 + t) * _L, _L)]
                b = plsc.bitcast(v, jnp.int32)
                col = lax.bitwise_and(lax.shift_right_logical(b, sh_idx),
                                      m127)
                row = lax.bitwise_and(
                    lax.shift_right_logical(b, sh_idx + 7), rmask)
                if dual:
                    p = lax.bitwise_and(lax.shift_right_logical(b, sh_pref),
                                        pmask)
                    plsc.addupdate_scatter(hist, [row, col], ones,
                                           mask=(p == pa))
                    plsc.addupdate_scatter(hist, [row + off_b, col], ones,
                                           mask=(p == pb))
                else:
                    plsc.addupdate_scatter(hist, [row, col], ones)
            return cc

        lax.fori_loop(0, _BLK // (8 * _L), g_body, 0)

    def blk_body(j, c):
        i0 = 2 * j
        issue(i0 + 1, buf1, sem1)
        drain(buf0, sem0)
        consume(buf0)

        @pl.when(i0 + 2 < n_blk)
        def _():
            issue(i0 + 2, buf0, sem0)

        drain(buf1, sem1)
        consume(buf1)
        return c

    lax.fori_loop(0, n_blk // 2, blk_body, 0)
    pltpu.sync_copy(hist, out_hbm.at[wid])


def _make_hist(n_total, b_bins, sh_pref, sh_idx, dual):
    mesh = plsc.VectorSubcoreMesh(core_axis_name="c", subcore_axis_name="s",
                                  num_cores=_NC, num_subcores=_NS)
    nbt = (2 if dual else 1) * b_bins
    body = functools.partial(_hist_body, n_total=n_total, b_bins=b_bins,
                             sh_pref=sh_pref, sh_idx=sh_idx, dual=dual)
    return pl.kernel(
        body,
        out_type=jax.ShapeDtypeStruct((_NW, nbt // 128, 128), jnp.int32),
        mesh=mesh,
        compiler_params=pltpu.CompilerParams(needs_layout_passes=False),
        scratch_types=[
            pltpu.VMEM((_BLK,), jnp.float32),
            pltpu.VMEM((_BLK,), jnp.float32),
            pltpu.VMEM((nbt // 128, 128), jnp.int32),
            pltpu.VMEM((2 * _L,), jnp.int32),
            pltpu.SemaphoreType.DMA,
            pltpu.SemaphoreType.DMA,
        ],
    )


def _advance(g, rank):
    c = jnp.cumsum(g)
    b = jnp.searchsorted(c, rank, side="right").astype(jnp.int32)
    return b, rank - (c[b] - g[b])


def _mask_kernel(t_ref, x_ref, o_ref):
    xv = x_ref[...]
    o_ref[...] = jnp.where(jnp.abs(xv) > t_ref[0, 0], xv, 0.0)


@jax.jit
def kernel(x):
    shape = x.shape
    n = x.size
    xf = x.reshape(n)

    pos = _RATIO * (n - 1)
    k = int(pos)
    frac = pos - k

    hist1 = _make_hist(n, 65536, 31, 15, False)
    hist2 = _make_hist(n, 32768, 15, 0, True)

    zero_pref = jnp.zeros((2 * _L,), jnp.int32)
    h1 = hist1(xf, zero_pref).reshape(_NW, 65536).sum(0)
    ba, ra = _advance(h1, jnp.int32(k))
    bb, rb = _advance(h1, jnp.int32(k + 1))

    pref2 = jnp.concatenate([jnp.full((_L,), ba), jnp.full((_L,), bb)])
    h2 = hist2(xf, pref2).reshape(_NW, 2, 32768).sum(0)
    b2a, _ = _advance(h2[0], ra)
    b2b, _ = _advance(h2[1], rb)

    va = lax.bitcast_convert_type((ba << 15) | b2a, jnp.float32)
    vb = lax.bitcast_convert_type((bb << 15) | b2b, jnp.float32)
    t = va + (vb - va) * jnp.float32(frac)

    rows = n // 2048
    blk_rows = 256
    out = pl.pallas_call(
        _mask_kernel,
        grid=(rows // blk_rows,),
        in_specs=[
            pl.BlockSpec(memory_space=pltpu.SMEM),
            pl.BlockSpec((blk_rows, 2048), lambda i: (i, 0)),
        ],
        out_specs=pl.BlockSpec((blk_rows, 2048), lambda i: (i, 0)),
        out_shape=jax.ShapeDtypeStruct((rows, 2048), jnp.float32),
    )(t.reshape(1, 1), x.reshape(rows, 2048))
    return out.reshape(shape)


# speedup vs baseline: 1.2232x; 1.2232x over previous
import functools

import jax
import jax.numpy as jnp
from jax import lax
from jax.experimental import pallas as pl
from jax.experimental.pallas import tpu as pltpu
from jax.experimental.pallas import tpu_sc as plsc

_RATIO = 0.5
_NC = 2
_NS = 16
_L = 16
_NW = _NC * _NS
_BLK = 8192


def _hist_body(x_hbm, pref_hbm, out_hbm, buf0, buf1, hist, pref_v,
               sem0, sem1, *, n_total, b_bins, sh_pref, sh_idx, dual):
    per_tile = n_total // _NW
    n_blk = per_tile // _BLK
    wid = lax.axis_index("s") * _NC + lax.axis_index("c")
    base = wid * per_tile
    nbt = (2 if dual else 1) * b_bins

    def issue(i, buf, sem):
        pltpu.async_copy(x_hbm.at[pl.ds(base + i * _BLK, _BLK)], buf, sem)

    def drain(buf, sem):
        pltpu.make_async_copy(x_hbm.at[pl.ds(base, _BLK)], buf, sem).wait()

    issue(0, buf0, sem0)

    zeros16 = jnp.zeros((_L,), jnp.int32)
    col16 = lax.iota(jnp.int32, _L)

    def zero_body(r, c):
        rv = jnp.full((_L,), r, jnp.int32)
        for t in range(8):
            plsc.store_scatter(hist, [rv, col16 + t * _L], zeros16)
        return c

    lax.fori_loop(0, nbt // 128, zero_body, 0)

    pltpu.sync_copy(pref_hbm, pref_v)
    pa = pref_v[pl.ds(0, _L)]
    pb = pref_v[pl.ds(_L, _L)]

    ones = jnp.ones((_L,), jnp.int32)
    m127 = jnp.int32(127)
    rmask = jnp.int32(b_bins // 128 - 1)
    pmask = jnp.int32(0xFFFF)
    off_b = jnp.int32(b_bins // 128)

    def consume(buf):
        def g_body(g, cc):
            for t in range(8):
                v = buf[pl.ds((g * 8 + t) * _L, _L)]
                b = plsc.bitcast(v, jnp.int32)
                col = lax.bitwise_and(lax.shift_right_logical(b, sh_idx),
                                      m127)
                row = lax.bitwise_and(
                    lax.shift_right_logical(b, sh_idx + 7), rmask)
                if dual:
                    p = lax.bitwise_and(lax.shift_right_logical(b, sh_pref),
                                        pmask)
                    plsc.addupdate_scatter(hist, [row, col], ones,
                                           mask=(p == pa))
                    plsc.addupdate_scatter(hist, [row + off_b, col], ones,
                                           mask=(p == pb))
                else:
                    plsc.addupdate_scatter(hist, [row, col], ones)
            return cc

        lax.fori_loop(0, _BLK // (8 * _L), g_body, 0)

    def blk_body(j, c):
        i0 = 2 * j
        issue(i0 + 1, buf1, sem1)
        drain(buf0, sem0)
        consume(buf0)

        @pl.when(i0 + 2 < n_blk)
        def _():
            issue(i0 + 2, buf0, sem0)

        drain(buf1, sem1)
        consume(buf1)
        return c

    lax.fori_loop(0, n_blk // 2, blk_body, 0)
    pltpu.sync_copy(hist, out_hbm.at[wid])


def _make_hist(n_total, b_bins, sh_pref, sh_idx, dual):
    mesh = plsc.VectorSubcoreMesh(core_axis_name="c", subcore_axis_name="s",
                                  num_cores=_NC, num_subcores=_NS)
    nbt = (2 if dual else 1) * b_bins
    body = functools.partial(_hist_body, n_total=n_total, b_bins=b_bins,
                             sh_pref=sh_pref, sh_idx=sh_idx, dual=dual)
    return pl.kernel(
        body,
        out_type=jax.ShapeDtypeStruct((_NW, nbt // 128, 128), jnp.int32),
        mesh=mesh,
        compiler_params=pltpu.CompilerParams(needs_layout_passes=False),
        scratch_types=[
            pltpu.VMEM((_BLK,), jnp.float32),
            pltpu.VMEM((_BLK,), jnp.float32),
            pltpu.VMEM((nbt // 128, 128), jnp.int32),
            pltpu.VMEM((2 * _L,), jnp.int32),
            pltpu.SemaphoreType.DMA,
            pltpu.SemaphoreType.DMA,
        ],
    )


def _select_2d(hsum, rank):
    nr = hsum.shape[0]
    ri = lax.broadcasted_iota(jnp.int32, (nr, 1), 0).astype(jnp.float32)
    tri_r = (lax.broadcasted_iota(jnp.int32, (nr, nr), 0) >=
             lax.broadcasted_iota(jnp.int32, (nr, nr), 1)).astype(jnp.float32)
    rowsum = jnp.sum(hsum, axis=1, keepdims=True)
    s = jnp.sum(jnp.dot(tri_r, hsum,
                        precision=lax.Precision.HIGHEST),
                axis=1, keepdims=True)
    r = jnp.sum((s <= rank).astype(jnp.float32))
    base = jnp.sum(jnp.where(ri < r, rowsum, 0.0))
    rowvec = jnp.sum(jnp.where(ri == r, hsum, 0.0), axis=0,
                     keepdims=True)
    ci = lax.broadcasted_iota(jnp.int32, (1, 128), 1).astype(jnp.float32)
    tri_c = (lax.broadcasted_iota(jnp.int32, (128, 128), 0) <=
             lax.broadcasted_iota(jnp.int32, (128, 128), 1)).astype(jnp.float32)
    cc = jnp.dot(rowvec, tri_c,
                 precision=lax.Precision.HIGHEST)
    rank2 = rank - base
    col = jnp.sum((cc <= rank2).astype(jnp.float32))
    cc_at = jnp.sum(jnp.where(ci == col, cc, 0.0))
    cnt_at = jnp.sum(jnp.where(ci == col, rowvec, 0.0))
    return r * 128.0 + col, rank2 - (cc_at - cnt_at)


def _glue1_kernel(h_ref, pref_ref, ranks_ref, *, k):
    h = jnp.sum(h_ref[...].astype(jnp.float32), axis=0)
    ba, ra = _select_2d(h, jnp.float32(k))
    bb, rb = _select_2d(h, jnp.float32(k + 1))
    ba_i = ba.astype(jnp.int32)
    bb_i = bb.astype(jnp.int32)
    for i in range(_L):
        pref_ref[i] = ba_i
        pref_ref[_L + i] = bb_i
    ranks_ref[0] = ra.astype(jnp.int32)
    ranks_ref[1] = rb.astype(jnp.int32)


def _glue2_kernel(h_ref, pref_ref, ranks_ref, t_ref, *, frac):
    h = jnp.sum(h_ref[...].astype(jnp.float32), axis=0)
    b2a, _ = _select_2d(h[:256], ranks_ref[0].astype(jnp.float32))
    b2b, _ = _select_2d(h[256:], ranks_ref[1].astype(jnp.float32))
    bits_a = pref_ref[0] * 32768 + b2a.astype(jnp.int32)
    bits_b = pref_ref[_L] * 32768 + b2b.astype(jnp.int32)
    va = lax.bitcast_convert_type(bits_a, jnp.float32)
    vb = lax.bitcast_convert_type(bits_b, jnp.float32)
    t_ref[0] = va + (vb - va) * jnp.float32(frac)


def _mask_kernel(t_ref, x_ref, o_ref):
    xv = x_ref[...]
    o_ref[...] = jnp.where(jnp.abs(xv) > t_ref[0, 0], xv, 0.0)


@jax.jit
def kernel(x):
    shape = x.shape
    n = x.size
    xf = x.reshape(n)

    pos = _RATIO * (n - 1)
    k = int(pos)
    frac = pos - k

    hist1 = _make_hist(n, 65536, 31, 15, False)
    hist2 = _make_hist(n, 32768, 15, 0, True)

    zero_pref = jnp.zeros((2 * _L,), jnp.int32)
    h1raw = hist1(xf, zero_pref)

    pref2, ranks = pl.pallas_call(
        functools.partial(_glue1_kernel, k=k),
        in_specs=[pl.BlockSpec(memory_space=pltpu.VMEM)],
        out_specs=[pl.BlockSpec(memory_space=pltpu.SMEM),
                   pl.BlockSpec(memory_space=pltpu.SMEM)],
        out_shape=[jax.ShapeDtypeStruct((2 * _L,), jnp.int32),
                   jax.ShapeDtypeStruct((2,), jnp.int32)],
    )(h1raw)

    h2raw = hist2(xf, pref2)
    t = pl.pallas_call(
        functools.partial(_glue2_kernel, frac=frac),
        in_specs=[pl.BlockSpec(memory_space=pltpu.VMEM),
                  pl.BlockSpec(memory_space=pltpu.SMEM),
                  pl.BlockSpec(memory_space=pltpu.SMEM)],
        out_specs=pl.BlockSpec(memory_space=pltpu.SMEM),
        out_shape=jax.ShapeDtypeStruct((1,), jnp.float32),
    )(h2raw, pref2, ranks)

    rows = n // 2048
    blk_rows = 256
    out = pl.pallas_call(
        _mask_kernel,
        grid=(rows // blk_rows,),
        in_specs=[
            pl.BlockSpec(memory_space=pltpu.SMEM),
            pl.BlockSpec((blk_rows, 2048), lambda i: (i, 0)),
        ],
        out_specs=pl.BlockSpec((blk_rows, 2048), lambda i: (i, 0)),
        out_shape=jax.ShapeDtypeStruct((rows, 2048), jnp.float32),
    )(t.reshape(1, 1), x.reshape(rows, 2048))
    return out.reshape(shape)


# speedup vs baseline: 2.8611x; 2.3390x over previous
import functools

import jax
import jax.numpy as jnp
from jax import lax
from jax.experimental import pallas as pl
from jax.experimental.pallas import tpu as pltpu
from jax.experimental.pallas import tpu_sc as plsc

_RATIO = 0.5
_NC = 2
_NS = 16
_L = 16
_NW = _NC * _NS
_BLK = 8192


def _hist_body(x_hbm, pref_hbm, out_hbm, buf0, buf1, hist, pref_v,
               sem0, sem1, *, n_total, b_bins, sh_pref, sh_idx, dual):
    per_tile = n_total // _NW
    n_blk = per_tile // _BLK
    wid = lax.axis_index("s") * _NC + lax.axis_index("c")
    base = wid * per_tile
    nbt = (2 if dual else 1) * b_bins

    def issue(i, buf, sem):
        pltpu.async_copy(x_hbm.at[pl.ds(base + i * _BLK, _BLK)], buf, sem)

    def drain(buf, sem):
        pltpu.make_async_copy(x_hbm.at[pl.ds(base, _BLK)], buf, sem).wait()

    issue(0, buf0, sem0)

    zeros16 = jnp.zeros((_L,), jnp.int32)
    col16 = lax.iota(jnp.int32, _L)

    @plsc.parallel_loop(0, nbt // 128, 1, unroll=4)
    def zero_body(r):
        rv = jnp.full((_L,), r, jnp.int32)
        for t in range(8):
            plsc.store_scatter(hist, [rv, col16 + t * _L], zeros16)

    pltpu.sync_copy(pref_hbm, pref_v)
    pa = pref_v[pl.ds(0, _L)]
    pb = pref_v[pl.ds(_L, _L)]

    ones = jnp.ones((_L,), jnp.int32)
    m127 = jnp.int32(127)
    rmask = jnp.int32(b_bins // 128 - 1)
    pmask = jnp.int32(0xFFFF)
    off_b = jnp.int32(b_bins // 128)

    def consume(buf):
        @plsc.parallel_loop(0, _BLK // _L, 1, unroll=8)
        def g_body(g):
            v = buf[pl.ds(g * _L, _L)]
            b = plsc.bitcast(v, jnp.int32)
            col = lax.bitwise_and(lax.shift_right_logical(b, sh_idx), m127)
            row = lax.bitwise_and(
                lax.shift_right_logical(b, sh_idx + 7), rmask)
            if dual:
                p = lax.bitwise_and(lax.shift_right_logical(b, sh_pref),
                                    pmask)
                plsc.addupdate_scatter(hist, [row, col], ones,
                                       mask=(p == pa))
                plsc.addupdate_scatter(hist, [row + off_b, col], ones,
                                       mask=(p == pb))
            else:
                plsc.addupdate_scatter(hist, [row, col], ones)

    def blk_body(j, c):
        i0 = 2 * j
        issue(i0 + 1, buf1, sem1)
        drain(buf0, sem0)
        consume(buf0)

        @pl.when(i0 + 2 < n_blk)
        def _():
            issue(i0 + 2, buf0, sem0)

        drain(buf1, sem1)
        consume(buf1)
        return c

    lax.fori_loop(0, n_blk // 2, blk_body, 0)
    pltpu.sync_copy(hist, out_hbm.at[wid])


def _make_hist(n_total, b_bins, sh_pref, sh_idx, dual):
    mesh = plsc.VectorSubcoreMesh(core_axis_name="c", subcore_axis_name="s",
                                  num_cores=_NC, num_subcores=_NS)
    nbt = (2 if dual else 1) * b_bins
    body = functools.partial(_hist_body, n_total=n_total, b_bins=b_bins,
                             sh_pref=sh_pref, sh_idx=sh_idx, dual=dual)
    return pl.kernel(
        body,
        out_type=jax.ShapeDtypeStruct((_NW, nbt // 128, 128), jnp.int32),
        mesh=mesh,
        compiler_params=pltpu.CompilerParams(needs_layout_passes=False),
        scratch_types=[
            pltpu.VMEM((_BLK,), jnp.float32),
            pltpu.VMEM((_BLK,), jnp.float32),
            pltpu.VMEM((nbt // 128, 128), jnp.int32),
            pltpu.VMEM((2 * _L,), jnp.int32),
            pltpu.SemaphoreType.DMA,
            pltpu.SemaphoreType.DMA,
        ],
    )


def _select_2d(hsum, rank):
    nr = hsum.shape[0]
    ri = lax.broadcasted_iota(jnp.int32, (nr, 1), 0).astype(jnp.float32)
    tri_r = (lax.broadcasted_iota(jnp.int32, (nr, nr), 0) >=
             lax.broadcasted_iota(jnp.int32, (nr, nr), 1)).astype(jnp.float32)
    rowsum = jnp.sum(hsum, axis=1, keepdims=True)
    s = jnp.sum(jnp.dot(tri_r, hsum,
                        precision=lax.Precision.HIGHEST),
                axis=1, keepdims=True)
    r = jnp.sum((s <= rank).astype(jnp.float32))
    base = jnp.sum(jnp.where(ri < r, rowsum, 0.0))
    rowvec = jnp.sum(jnp.where(ri == r, hsum, 0.0), axis=0,
                     keepdims=True)
    ci = lax.broadcasted_iota(jnp.int32, (1, 128), 1).astype(jnp.float32)
    tri_c = (lax.broadcasted_iota(jnp.int32, (128, 128), 0) <=
             lax.broadcasted_iota(jnp.int32, (128, 128), 1)).astype(jnp.float32)
    cc = jnp.dot(rowvec, tri_c,
                 precision=lax.Precision.HIGHEST)
    rank2 = rank - base
    col = jnp.sum((cc <= rank2).astype(jnp.float32))
    cc_at = jnp.sum(jnp.where(ci == col, cc, 0.0))
    cnt_at = jnp.sum(jnp.where(ci == col, rowvec, 0.0))
    return r * 128.0 + col, rank2 - (cc_at - cnt_at)


def _glue1_kernel(h_ref, pref_ref, ranks_ref, *, k):
    h = jnp.sum(h_ref[...].astype(jnp.float32), axis=0)
    ba, ra = _select_2d(h, jnp.float32(k))
    bb, rb = _select_2d(h, jnp.float32(k + 1))
    ba_i = ba.astype(jnp.int32)
    bb_i = bb.astype(jnp.int32)
    for i in range(_L):
        pref_ref[i] = ba_i
        pref_ref[_L + i] = bb_i
    ranks_ref[0] = ra.astype(jnp.int32)
    ranks_ref[1] = rb.astype(jnp.int32)


def _glue2_kernel(h_ref, pref_ref, ranks_ref, t_ref, *, frac):
    h = jnp.sum(h_ref[...].astype(jnp.float32), axis=0)
    b2a, _ = _select_2d(h[:256], ranks_ref[0].astype(jnp.float32))
    b2b, _ = _select_2d(h[256:], ranks_ref[1].astype(jnp.float32))
    bits_a = pref_ref[0] * 32768 + b2a.astype(jnp.int32)
    bits_b = pref_ref[_L] * 32768 + b2b.astype(jnp.int32)
    va = lax.bitcast_convert_type(bits_a, jnp.float32)
    vb = lax.bitcast_convert_type(bits_b, jnp.float32)
    t_ref[0] = va + (vb - va) * jnp.float32(frac)


def _mask_kernel(t_ref, x_ref, o_ref):
    xv = x_ref[...]
    o_ref[...] = jnp.where(jnp.abs(xv) > t_ref[0, 0], xv, 0.0)


@jax.jit
def kernel(x):
    shape = x.shape
    n = x.size
    xf = x.reshape(n)

    pos = _RATIO * (n - 1)
    k = int(pos)
    frac = pos - k

    hist1 = _make_hist(n, 65536, 31, 15, False)
    hist2 = _make_hist(n, 32768, 15, 0, True)

    zero_pref = jnp.zeros((2 * _L,), jnp.int32)
    h1raw = hist1(xf, zero_pref)

    pref2, ranks = pl.pallas_call(
        functools.partial(_glue1_kernel, k=k),
        in_specs=[pl.BlockSpec(memory_space=pltpu.VMEM)],
        out_specs=[pl.BlockSpec(memory_space=pltpu.SMEM),
                   pl.BlockSpec(memory_space=pltpu.SMEM)],
        out_shape=[jax.ShapeDtypeStruct((2 * _L,), jnp.int32),
                   jax.ShapeDtypeStruct((2,), jnp.int32)],
    )(h1raw)

    h2raw = hist2(xf, pref2)
    t = pl.pallas_call(
        functools.partial(_glue2_kernel, frac=frac),
        in_specs=[pl.BlockSpec(memory_space=pltpu.VMEM),
                  pl.BlockSpec(memory_space=pltpu.SMEM),
                  pl.BlockSpec(memory_space=pltpu.SMEM)],
        out_specs=pl.BlockSpec(memory_space=pltpu.SMEM),
        out_shape=jax.ShapeDtypeStruct((1,), jnp.float32),
    )(h2raw, pref2, ranks)

    rows = n // 2048
    blk_rows = 256
    out = pl.pallas_call(
        _mask_kernel,
        grid=(rows // blk_rows,),
        in_specs=[
            pl.BlockSpec(memory_space=pltpu.SMEM),
            pl.BlockSpec((blk_rows, 2048), lambda i: (i, 0)),
        ],
        out_specs=pl.BlockSpec((blk_rows, 2048), lambda i: (i, 0)),
        out_shape=jax.ShapeDtypeStruct((rows, 2048), jnp.float32),
    )(t.reshape(1, 1), x.reshape(rows, 2048))
    return out.reshape(shape)


# speedup vs baseline: 3.6654x; 1.2811x over previous
import functools

import jax
import jax.numpy as jnp
from jax import lax
from jax.experimental import pallas as pl
from jax.experimental.pallas import tpu as pltpu
from jax.experimental.pallas import tpu_sc as plsc

_RATIO = 0.5
_NC = 2
_NS = 16
_L = 16
_NW = _NC * _NS


def _hist_body(x_hbm, pref_hbm, out_hbm, buf0, buf1, hist, pref_v,
               sem0, sem1, *, n_total, b_bins, sh_pref, sh_idx, dual):
    n_rows = n_total // 2048
    rows_per_tile = n_rows // _NW
    n_blk = rows_per_tile // 8
    wid = lax.axis_index("s") * _NC + lax.axis_index("c")
    base = wid * rows_per_tile
    nbt = (2 if dual else 1) * b_bins

    def issue(i, buf, sem):
        pltpu.async_copy(x_hbm.at[pl.ds(base + i * 8, 8)], buf, sem)

    def drain(buf, sem):
        pltpu.make_async_copy(x_hbm.at[pl.ds(base, 8)], buf, sem).wait()

    issue(0, buf0, sem0)

    zeros16 = jnp.zeros((_L,), jnp.int32)
    col16 = lax.iota(jnp.int32, _L)

    @plsc.parallel_loop(0, nbt // 128, 1, unroll=4)
    def zero_body(r):
        rv = jnp.full((_L,), r, jnp.int32)
        for t in range(8):
            plsc.store_scatter(hist, [rv, col16 + t * _L], zeros16)

    pltpu.sync_copy(pref_hbm, pref_v)
    pa = pref_v[pl.ds(0, _L)]
    pb = pref_v[pl.ds(_L, _L)]

    ones = jnp.ones((_L,), jnp.int32)
    m127 = jnp.int32(127)
    rmask = jnp.int32(b_bins // 128 - 1)
    pmask = jnp.int32(0xFFFF)
    off_b = jnp.int32(b_bins // 128)

    def consume(buf):
        @plsc.parallel_loop(0, (8 * 2048) // _L, 1, unroll=8)
        def g_body(g):
            v = buf[g >> 7, pl.ds((g & 127) * _L, _L)]
            b = plsc.bitcast(v, jnp.int32)
            col = lax.bitwise_and(lax.shift_right_logical(b, sh_idx), m127)
            row = lax.bitwise_and(
                lax.shift_right_logical(b, sh_idx + 7), rmask)
            if dual:
                p = lax.bitwise_and(lax.shift_right_logical(b, sh_pref),
                                    pmask)
                plsc.addupdate_scatter(hist, [row, col], ones,
                                       mask=(p == pa))
                plsc.addupdate_scatter(hist, [row + off_b, col], ones,
                                       mask=(p == pb))
            else:
                plsc.addupdate_scatter(hist, [row, col], ones)

    def blk_body(j, c):
        i0 = 2 * j
        issue(i0 + 1, buf1, sem1)
        drain(buf0, sem0)
        consume(buf0)

        @pl.when(i0 + 2 < n_blk)
        def _():
            issue(i0 + 2, buf0, sem0)

        drain(buf1, sem1)
        consume(buf1)
        return c

    lax.fori_loop(0, n_blk // 2, blk_body, 0)
    pltpu.sync_copy(hist, out_hbm.at[wid])


def _make_hist(n_total, b_bins, sh_pref, sh_idx, dual):
    mesh = plsc.VectorSubcoreMesh(core_axis_name="c", subcore_axis_name="s",
                                  num_cores=_NC, num_subcores=_NS)
    nbt = (2 if dual else 1) * b_bins
    body = functools.partial(_hist_body, n_total=n_total, b_bins=b_bins,
                             sh_pref=sh_pref, sh_idx=sh_idx, dual=dual)
    return pl.kernel(
        body,
        out_type=jax.ShapeDtypeStruct((_NW, nbt // 128, 128), jnp.int32),
        mesh=mesh,
        compiler_params=pltpu.CompilerParams(needs_layout_passes=False),
        scratch_types=[
            pltpu.VMEM((8, 2048), jnp.float32),
            pltpu.VMEM((8, 2048), jnp.float32),
            pltpu.VMEM((nbt // 128, 128), jnp.int32),
            pltpu.VMEM((2 * _L,), jnp.int32),
            pltpu.SemaphoreType.DMA,
            pltpu.SemaphoreType.DMA,
        ],
    )


def _select_2d(hsum, rank):
    nr = hsum.shape[0]
    ri = lax.broadcasted_iota(jnp.int32, (nr, 1), 0).astype(jnp.float32)
    tri_r = (lax.broadcasted_iota(jnp.int32, (nr, nr), 0) >=
             lax.broadcasted_iota(jnp.int32, (nr, nr), 1)).astype(jnp.float32)
    rowsum = jnp.sum(hsum, axis=1, keepdims=True)
    s = jnp.sum(jnp.dot(tri_r, hsum,
                        precision=lax.Precision.HIGHEST),
                axis=1, keepdims=True)
    r = jnp.sum((s <= rank).astype(jnp.float32))
    base = jnp.sum(jnp.where(ri < r, rowsum, 0.0))
    rowvec = jnp.sum(jnp.where(ri == r, hsum, 0.0), axis=0,
                     keepdims=True)
    ci = lax.broadcasted_iota(jnp.int32, (1, 128), 1).astype(jnp.float32)
    tri_c = (lax.broadcasted_iota(jnp.int32, (128, 128), 0) <=
             lax.broadcasted_iota(jnp.int32, (128, 128), 1)).astype(jnp.float32)
    cc = jnp.dot(rowvec, tri_c,
                 precision=lax.Precision.HIGHEST)
    rank2 = rank - base
    col = jnp.sum((cc <= rank2).astype(jnp.float32))
    cc_at = jnp.sum(jnp.where(ci == col, cc, 0.0))
    cnt_at = jnp.sum(jnp.where(ci == col, rowvec, 0.0))
    return r * 128.0 + col, rank2 - (cc_at - cnt_at)


def _glue1_kernel(h_ref, pref_ref, ranks_ref, *, k):
    h = jnp.sum(h_ref[...].astype(jnp.float32), axis=0)
    ba, ra = _select_2d(h, jnp.float32(k))
    bb, rb = _select_2d(h, jnp.float32(k + 1))
    ba_i = ba.astype(jnp.int32)
    bb_i = bb.astype(jnp.int32)
    for i in range(_L):
        pref_ref[i] = ba_i
        pref_ref[_L + i] = bb_i
    ranks_ref[0] = ra.astype(jnp.int32)
    ranks_ref[1] = rb.astype(jnp.int32)


def _glue2_kernel(h_ref, pref_ref, ranks_ref, t_ref, *, frac):
    h = jnp.sum(h_ref[...].astype(jnp.float32), axis=0)
    b2a, _ = _select_2d(h[:256], ranks_ref[0].astype(jnp.float32))
    b2b, _ = _select_2d(h[256:], ranks_ref[1].astype(jnp.float32))
    bits_a = pref_ref[0] * 32768 + b2a.astype(jnp.int32)
    bits_b = pref_ref[_L] * 32768 + b2b.astype(jnp.int32)
    va = lax.bitcast_convert_type(bits_a, jnp.float32)
    vb = lax.bitcast_convert_type(bits_b, jnp.float32)
    t_ref[0] = va + (vb - va) * jnp.float32(frac)


def _mask_kernel(t_ref, x_ref, o_ref):
    xv = x_ref[...]
    o_ref[...] = jnp.where(jnp.abs(xv) > t_ref[0, 0], xv, 0.0)


@jax.jit
def kernel(x):
    shape = x.shape
    n = x.size
    x2d = x.reshape(n // 2048, 2048)

    pos = _RATIO * (n - 1)
    k = int(pos)
    frac = pos - k

    hist1 = _make_hist(n, 65536, 31, 15, False)
    hist2 = _make_hist(n, 32768, 15, 0, True)

    zero_pref = jnp.zeros((2 * _L,), jnp.int32)
    h1raw = hist1(x2d, zero_pref)

    pref2, ranks = pl.pallas_call(
        functools.partial(_glue1_kernel, k=k),
        in_specs=[pl.BlockSpec(memory_space=pltpu.VMEM)],
        out_specs=[pl.BlockSpec(memory_space=pltpu.SMEM),
                   pl.BlockSpec(memory_space=pltpu.SMEM)],
        out_shape=[jax.ShapeDtypeStruct((2 * _L,), jnp.int32),
                   jax.ShapeDtypeStruct((2,), jnp.int32)],
    )(h1raw)

    h2raw = hist2(x2d, pref2)
    t = pl.pallas_call(
        functools.partial(_glue2_kernel, frac=frac),
        in_specs=[pl.BlockSpec(memory_space=pltpu.VMEM),
                  pl.BlockSpec(memory_space=pltpu.SMEM),
                  pl.BlockSpec(memory_space=pltpu.SMEM)],
        out_specs=pl.BlockSpec(memory_space=pltpu.SMEM),
        out_shape=jax.ShapeDtypeStruct((1,), jnp.float32),
    )(h2raw, pref2, ranks)

    rows = n // 2048
    blk_rows = 256
    out = pl.pallas_call(
        _mask_kernel,
        grid=(rows // blk_rows,),
        in_specs=[
            pl.BlockSpec(memory_space=pltpu.SMEM),
            pl.BlockSpec((blk_rows, 2048), lambda i: (i, 0)),
        ],
        out_specs=pl.BlockSpec((blk_rows, 2048), lambda i: (i, 0)),
        out_shape=jax.ShapeDtypeStruct((rows, 2048), jnp.float32),
    )(t.reshape(1, 1), x.reshape(rows, 2048))
    return out.reshape(shape)
